# Initial kernel scaffold; baseline (speedup 1.0000x reference)
import jax
import jax.numpy as jnp
from jax.experimental import pallas as pl


def kernel(x, edge_index, W1l, b1, W1r, W2l, b2, W2r):
    raise NotImplementedError("write your pallas kernel here")


# speedup vs baseline: 10.7542x; 10.7542x over previous
import functools

import jax
import jax.numpy as jnp
from jax import lax
from jax.experimental import pallas as pl
from jax.experimental.pallas import tpu as pltpu
from jax.experimental.pallas import tpu_sc as plsc

_N = 10000
_E = 320000
_DIN = 128
_DH = 32
_NP = 10240

_NC = 2
_NS = 16
_NW = _NC * _NS
_EPW = _E // _NW
_CH = 128
_NFULL = _EPW // _CH
_TAIL = _EPW - _NFULL * _CH
_RPT = _NP // _NS


def _lin_body(x_ref, wl_ref, wr_ref, b_ref, y_ref, z_ref):
    x = x_ref[...]
    y_ref[...] = jnp.dot(x, wl_ref[...], preferred_element_type=jnp.float32)
    z_ref[...] = (
        jnp.dot(x, wr_ref[...], preferred_element_type=jnp.float32) + b_ref[...]
    )


def _mid_body(p_ref, cp_ref, z1_ref, w2l_ref, w2r_ref, b2_ref, y2_ref, z2_ref):
    agg = p_ref[0] + p_ref[1]
    cnt = jnp.maximum(cp_ref[0] + cp_ref[1], 1.0)
    h = jnp.maximum(agg / cnt[:, None] + z1_ref[...], 0.0)
    y2_ref[...] = jnp.dot(h, w2l_ref[...], preferred_element_type=jnp.float32)
    z2_ref[...] = (
        jnp.dot(h, w2r_ref[...], preferred_element_type=jnp.float32) + b2_ref[...]
    )


def _fin_body(q_ref, cp_ref, z2_ref, o_ref):
    cnt = jnp.maximum(cp_ref[0] + cp_ref[1], 1.0)
    o_ref[...] = (q_ref[0] + q_ref[1]) / cnt[:, None] + z2_ref[...]


def _tc_call(body, out_shapes, *args):
    return pl.pallas_call(
        body,
        out_shape=out_shapes,
    )(*args)


def _edge_pass_body(with_count, *refs):
    if with_count:
        (y_hbm, src_hbm, dst_hbm, out_hbm, cnt_hbm,
         ysh, acc, cntacc, sidx, didx, rows, ones, zbuf, zcnt,
         sidx_t, didx_t, rows_t, ones_t, gsem) = refs
    else:
        (y_hbm, src_hbm, dst_hbm, out_hbm,
         ysh, acc, sidx, didx, rows, zbuf,
         sidx_t, didx_t, rows_t, gsem) = refs

    cid = lax.axis_index("c")
    tid = lax.axis_index("s")
    wid = tid * _NC + cid
    r0 = tid * _RPT

    pltpu.sync_copy(y_hbm.at[pl.ds(r0, _RPT)], zbuf)
    pltpu.sync_copy(zbuf, ysh.at[pl.ds(r0, _RPT)])

    zeros16 = jnp.zeros((16,), jnp.float32)

    def _zrow(i, _):
        zbuf[i, pl.ds(0, 16)] = zeros16
        zbuf[i, pl.ds(16, 16)] = zeros16
        return 0
    lax.fori_loop(0, _RPT, _zrow, 0)

    if with_count:
        def _zcnt(i, _):
            zcnt[pl.ds(i * 16, 16)] = zeros16
            return 0
        lax.fori_loop(0, _RPT // 16, _zcnt, 0)
        ones16 = jnp.ones((16,), jnp.float32)
        for i in range(_CH // 16):
            ones[pl.ds(i * 16, 16)] = ones16
        ones_t[...] = ones16

    pltpu.sync_copy(zbuf, acc.at[pl.ds(r0, _RPT)])
    if with_count:
        pltpu.sync_copy(zcnt, cntacc.at[pl.ds(r0, _RPT)])
    plsc.subcore_barrier()

    ebase = wid * _EPW

    def _chunk(i, _):
        base = ebase + i * _CH
        pltpu.sync_copy(src_hbm.at[pl.ds(base, _CH)], sidx)
        pltpu.sync_copy(dst_hbm.at[pl.ds(base, _CH)], didx)
        pltpu.async_copy(ysh.at[sidx], rows, gsem).wait()
        pltpu.sync_copy(rows, acc.at[didx], add=True)
        if with_count:
            pltpu.sync_copy(ones, cntacc.at[didx], add=True)
        return 0
    lax.fori_loop(0, _NFULL, _chunk, 0)

    tbase = ebase + _NFULL * _CH
    pltpu.sync_copy(src_hbm.at[pl.ds(tbase, _TAIL)], sidx_t)
    pltpu.sync_copy(dst_hbm.at[pl.ds(tbase, _TAIL)], didx_t)
    pltpu.async_copy(ysh.at[sidx_t], rows_t, gsem).wait()
    pltpu.sync_copy(rows_t, acc.at[didx_t], add=True)
    if with_count:
        pltpu.sync_copy(ones_t, cntacc.at[didx_t], add=True)

    plsc.subcore_barrier()

    pltpu.sync_copy(acc.at[pl.ds(r0, _RPT)], zbuf)
    pltpu.sync_copy(zbuf, out_hbm.at[cid, pl.ds(r0, _RPT)])
    if with_count:
        pltpu.sync_copy(cntacc.at[pl.ds(r0, _RPT)], zcnt)
        pltpu.sync_copy(zcnt, cnt_hbm.at[cid, pl.ds(r0, _RPT)])


def _make_edge_pass(with_count):
    out_type = [jax.ShapeDtypeStruct((_NC, _NP, _DH), jnp.float32)]
    scratch = [
        pltpu.VMEM_SHARED((_NP, _DH), jnp.float32),
        pltpu.VMEM_SHARED((_NP, _DH), jnp.float32),
    ]
    if with_count:
        out_type.append(jax.ShapeDtypeStruct((_NC, _NP), jnp.float32))
        scratch.append(pltpu.VMEM_SHARED((_NP,), jnp.float32))
    scratch += [
        pltpu.VMEM((_CH,), jnp.int32),
        pltpu.VMEM((_CH,), jnp.int32),
        pltpu.VMEM((_CH, _DH), jnp.float32),
    ]
    if with_count:
        scratch.append(pltpu.VMEM((_CH,), jnp.float32))
    scratch.append(pltpu.VMEM((_RPT, _DH), jnp.float32))
    if with_count:
        scratch.append(pltpu.VMEM((_RPT,), jnp.float32))
    scratch += [
        pltpu.VMEM((_TAIL,), jnp.int32),
        pltpu.VMEM((_TAIL,), jnp.int32),
        pltpu.VMEM((_TAIL, _DH), jnp.float32),
    ]
    if with_count:
        scratch.append(pltpu.VMEM((_TAIL,), jnp.float32))
    scratch.append(pltpu.SemaphoreType.DMA)

    return pl.kernel(
        functools.partial(_edge_pass_body, with_count),
        out_type=out_type,
        mesh=plsc.VectorSubcoreMesh(core_axis_name="c", subcore_axis_name="s"),
        scratch_types=scratch,
        compiler_params=pltpu.CompilerParams(use_tc_tiling_on_sc=False),
    )


_edge_pass_l1 = _make_edge_pass(True)
_edge_pass_l2 = _make_edge_pass(False)


def _impl(x, edge_index, W1l, b1, W1r, W2l, b2, W2r):
    xp = jnp.pad(x, ((0, _NP - _N), (0, 0)))
    ei = edge_index.astype(jnp.int32)
    src, dst = ei[0], ei[1]
    b1r = b1.reshape(1, _DH)
    b2r = b2.reshape(1, _DH)

    y1, z1 = _tc_call(
        _lin_body,
        [jax.ShapeDtypeStruct((_NP, _DH), jnp.float32),
         jax.ShapeDtypeStruct((_NP, _DH), jnp.float32)],
        xp, W1l, W1r, b1r,
    )
    p, cp = _edge_pass_l1(y1, src, dst)
    y2, z2 = _tc_call(
        _mid_body,
        [jax.ShapeDtypeStruct((_NP, _DH), jnp.float32),
         jax.ShapeDtypeStruct((_NP, _DH), jnp.float32)],
        p, cp, z1, W2l, W2r, b2r,
    )
    (q,) = _edge_pass_l2(y2, src, dst)
    out = _tc_call(
        _fin_body,
        jax.ShapeDtypeStruct((_NP, _DH), jnp.float32),
        q, cp, z2,
    )
    return out[:_N]


def kernel(x, edge_index, W1l, b1, W1r, W2l, b2, W2r):
    assert x.shape == (_N, _DIN) and edge_index.shape == (2, _E)
    return _impl(x, edge_index, W1l, b1, W1r, W2l, b2, W2r)


# speedup vs baseline: 21.8690x; 2.0335x over previous
import functools

import jax
import jax.numpy as jnp
from jax import lax
from jax.experimental import pallas as pl
from jax.experimental.pallas import tpu as pltpu
from jax.experimental.pallas import tpu_sc as plsc

_N = 10000
_E = 320000
_DIN = 128
_DH = 32
_NP = 10240

_NC = 2
_NS = 16
_NW = _NC * _NS
_CH = 128
_NCHUNK = _E // _CH
_CPW = _NCHUNK // _NW
_XTRA = _NCHUNK - _CPW * _NW
_NBUF = 6
_NGRP = _CPW // _NBUF
_RPT = _NP // _NS


def _lin_body(x_ref, wl_ref, wr_ref, b_ref, y_ref, z_ref):
    x = x_ref[...]
    y_ref[...] = jnp.dot(x, wl_ref[...], preferred_element_type=jnp.float32)
    z_ref[...] = (
        jnp.dot(x, wr_ref[...], preferred_element_type=jnp.float32) + b_ref[...]
    )


def _mid_body(p_ref, cp_ref, z1_ref, w2l_ref, w2r_ref, b2_ref, y2_ref, z2_ref):
    agg = p_ref[0] + p_ref[1]
    cnt = jnp.maximum(cp_ref[0] + cp_ref[1], 1.0)
    h = jnp.maximum(agg / cnt[:, None] + z1_ref[...], 0.0)
    y2_ref[...] = jnp.dot(h, w2l_ref[...], preferred_element_type=jnp.float32)
    z2_ref[...] = (
        jnp.dot(h, w2r_ref[...], preferred_element_type=jnp.float32) + b2_ref[...]
    )


def _fin_body(q_ref, cp_ref, z2_ref, o_ref):
    cnt = jnp.maximum(cp_ref[0] + cp_ref[1], 1.0)
    o_ref[...] = (q_ref[0] + q_ref[1]) / cnt[:, None] + z2_ref[...]


def _tc_call(body, out_shapes, *args):
    return pl.pallas_call(body, out_shape=out_shapes)(*args)


def _edge_pass_body(with_count, *refs):
    refs = list(refs)
    if with_count:
        (y_hbm, src_hbm, dst_hbm, out_hbm, cnt_hbm,
         ysh, acc, cntacc, sidx_all, didx_all) = refs[:10]
        rowbufs = refs[10:10 + _NBUF]
        (ystage, zbuf, zcnt, ones,
         psem, ssem) = refs[10 + _NBUF:16 + _NBUF]
        gsems = refs[16 + _NBUF:]
    else:
        (y_hbm, src_hbm, dst_hbm, out_hbm,
         ysh, acc, sidx_all, didx_all) = refs[:8]
        rowbufs = refs[8:8 + _NBUF]
        (ystage, zbuf, psem, ssem) = refs[8 + _NBUF:12 + _NBUF]
        gsems = refs[12 + _NBUF:]

    cid = lax.axis_index("c")
    tid = lax.axis_index("s")
    wid = tid * _NC + cid
    r0 = tid * _RPT
    c0 = wid * _CPW

    st = pltpu.async_copy(y_hbm.at[pl.ds(r0, _RPT)], ystage, psem)
    si = pltpu.async_copy(src_hbm.at[pl.ds(c0, _CPW)],
                          sidx_all.at[pl.ds(0, _CPW)], psem)
    di = pltpu.async_copy(dst_hbm.at[pl.ds(c0, _CPW)],
                          didx_all.at[pl.ds(0, _CPW)], psem)

    @pl.when(wid < _XTRA)
    def _():
        pltpu.async_copy(src_hbm.at[pl.ds(_CPW * _NW + wid, 1)],
                         sidx_all.at[pl.ds(_CPW, 1)], psem)
        pltpu.async_copy(dst_hbm.at[pl.ds(_CPW * _NW + wid, 1)],
                         didx_all.at[pl.ds(_CPW, 1)], psem)

    zeros16 = jnp.zeros((16,), jnp.float32)

    def _zrow(i, _):
        zbuf[i, pl.ds(0, 16)] = zeros16
        zbuf[i, pl.ds(16, 16)] = zeros16
        return 0
    lax.fori_loop(0, _RPT, _zrow, 0)

    if with_count:
        def _zc(i, _):
            zcnt[pl.ds(i * 16, 16)] = zeros16
            return 0
        lax.fori_loop(0, _RPT // 16, _zc, 0)
        ones16 = jnp.ones((16,), jnp.float32)
        for i in range(_CH // 16):
            ones[pl.ds(i * 16, 16)] = ones16

    pltpu.sync_copy(zbuf, acc.at[pl.ds(r0, _RPT)])
    if with_count:
        pltpu.sync_copy(zcnt, cntacc.at[pl.ds(r0, _RPT)])

    st.wait()
    pltpu.sync_copy(ystage, ysh.at[pl.ds(r0, _RPT)])
    si.wait()
    di.wait()

    @pl.when(wid < _XTRA)
    def _():
        pltpu.make_async_copy(src_hbm.at[pl.ds(_CPW * _NW + wid, 1)],
                              sidx_all.at[pl.ds(_CPW, 1)], psem).wait()
        pltpu.make_async_copy(dst_hbm.at[pl.ds(_CPW * _NW + wid, 1)],
                              didx_all.at[pl.ds(_CPW, 1)], psem).wait()

    plsc.subcore_barrier()

    def _group(g, _):
        c = c0 + g * _NBUF
        gds = [
            pltpu.async_copy(ysh.at[sidx_all.at[g * _NBUF + b]],
                             rowbufs[b], gsems[b])
            for b in range(_NBUF)
        ]
        sds = []
        for b in range(_NBUF):
            gds[b].wait()
            sds.append(pltpu.async_copy(
                rowbufs[b], acc.at[didx_all.at[g * _NBUF + b]], ssem,
                add=True))
            if with_count:
                sds.append(pltpu.async_copy(
                    ones, cntacc.at[didx_all.at[g * _NBUF + b]], ssem,
                    add=True))
        for d in sds:
            d.wait()
        return 0
    lax.fori_loop(0, _NGRP, _group, 0)

    @pl.when(wid < _XTRA)
    def _():
        pltpu.async_copy(ysh.at[sidx_all.at[_CPW]], rowbufs[0],
                         gsems[0]).wait()
        pltpu.async_copy(rowbufs[0], acc.at[didx_all.at[_CPW]], ssem,
                         add=True).wait()
        if with_count:
            pltpu.async_copy(ones, cntacc.at[didx_all.at[_CPW]], ssem,
                             add=True).wait()

    plsc.subcore_barrier()

    pltpu.sync_copy(acc.at[pl.ds(r0, _RPT)], zbuf)
    pltpu.sync_copy(zbuf, out_hbm.at[cid, pl.ds(r0, _RPT)])
    if with_count:
        pltpu.sync_copy(cntacc.at[pl.ds(r0, _RPT)], zcnt)
        pltpu.sync_copy(zcnt, cnt_hbm.at[cid, pl.ds(r0, _RPT)])


def _make_edge_pass(with_count):
    out_type = [jax.ShapeDtypeStruct((_NC, _NP, _DH), jnp.float32)]
    if with_count:
        out_type.append(jax.ShapeDtypeStruct((_NC, _NP), jnp.float32))
    scratch = [
        pltpu.VMEM_SHARED((_NP, _DH), jnp.float32),
        pltpu.VMEM_SHARED((_NP, _DH), jnp.float32),
    ]
    if with_count:
        scratch.append(pltpu.VMEM_SHARED((_NP,), jnp.float32))
    scratch += [
        pltpu.VMEM((_CPW + 1, _CH), jnp.int32),
        pltpu.VMEM((_CPW + 1, _CH), jnp.int32),
    ]
    scratch += [pltpu.VMEM((_CH, _DH), jnp.float32) for _ in range(_NBUF)]
    scratch.append(pltpu.VMEM((_RPT, _DH), jnp.float32))
    scratch.append(pltpu.VMEM((_RPT, _DH), jnp.float32))
    if with_count:
        scratch.append(pltpu.VMEM((_RPT,), jnp.float32))
        scratch.append(pltpu.VMEM((_CH,), jnp.float32))
    scratch += [pltpu.SemaphoreType.DMA,
                pltpu.SemaphoreType.DMA]
    scratch += [pltpu.SemaphoreType.DMA for _ in range(_NBUF)]

    return pl.kernel(
        functools.partial(_edge_pass_body, with_count),
        out_type=out_type,
        mesh=plsc.VectorSubcoreMesh(core_axis_name="c", subcore_axis_name="s"),
        scratch_types=scratch,
        compiler_params=pltpu.CompilerParams(use_tc_tiling_on_sc=False),
    )


_edge_pass_l1 = _make_edge_pass(True)
_edge_pass_l2 = _make_edge_pass(False)


def _impl(x, edge_index, W1l, b1, W1r, W2l, b2, W2r):
    xp = jnp.pad(x, ((0, _NP - _N), (0, 0)))
    ei = edge_index.astype(jnp.int32)
    src2 = ei[0].reshape(_NCHUNK, _CH)
    dst2 = ei[1].reshape(_NCHUNK, _CH)
    b1r = b1.reshape(1, _DH)
    b2r = b2.reshape(1, _DH)

    y1, z1 = _tc_call(
        _lin_body,
        [jax.ShapeDtypeStruct((_NP, _DH), jnp.float32),
         jax.ShapeDtypeStruct((_NP, _DH), jnp.float32)],
        xp, W1l, W1r, b1r,
    )
    p, cp = _edge_pass_l1(y1, src2, dst2)
    y2, z2 = _tc_call(
        _mid_body,
        [jax.ShapeDtypeStruct((_NP, _DH), jnp.float32),
         jax.ShapeDtypeStruct((_NP, _DH), jnp.float32)],
        p, cp, z1, W2l, W2r, b2r,
    )
    (q,) = _edge_pass_l2(y2, src2, dst2)
    out = _tc_call(
        _fin_body,
        jax.ShapeDtypeStruct((_NP, _DH), jnp.float32),
        q, cp, z2,
    )
    return out[:_N]


def kernel(x, edge_index, W1l, b1, W1r, W2l, b2, W2r):
    assert x.shape == (_N, _DIN) and edge_index.shape == (2, _E)
    return _impl(x, edge_index, W1l, b1, W1r, W2l, b2, W2r)


# speedup vs baseline: 23.0167x; 1.0525x over previous
import functools

import jax
import jax.numpy as jnp
from jax import lax
from jax.experimental import pallas as pl
from jax.experimental.pallas import tpu as pltpu
from jax.experimental.pallas import tpu_sc as plsc

_N = 10000
_E = 320000
_DIN = 128
_DH = 32
_NP = 10240

_NC = 2
_NS = 16
_NW = _NC * _NS
_CH = 128
_NCHUNK = _E // _CH
_CPW = _NCHUNK // _NW
_XTRA = _NCHUNK - _CPW * _NW
_BK = 3
_NGRP = _CPW // (2 * _BK)
_RPT = _NP // _NS

_BLK = 2048
_GRID = _NP // _BLK


def _lin_body(x_ref, wl_ref, wr_ref, b_ref, y_ref, z_ref):
    x = x_ref[...]
    y_ref[...] = jnp.dot(x, wl_ref[...], preferred_element_type=jnp.float32)
    z_ref[...] = (
        jnp.dot(x, wr_ref[...], preferred_element_type=jnp.float32) + b_ref[...]
    )


def _mid_body(p_ref, c0_ref, c1_ref, z1_ref, w2l_ref, w2r_ref, b2_ref,
              y2_ref, z2_ref):
    agg = p_ref[0] + p_ref[1]
    cnt = jnp.maximum(c0_ref[...] + c1_ref[...], 1.0)
    h = jnp.maximum(agg / cnt[:, None] + z1_ref[...], 0.0)
    y2_ref[...] = jnp.dot(h, w2l_ref[...], preferred_element_type=jnp.float32)
    z2_ref[...] = (
        jnp.dot(h, w2r_ref[...], preferred_element_type=jnp.float32) + b2_ref[...]
    )


def _fin_body(q_ref, c0_ref, c1_ref, z2_ref, o_ref):
    cnt = jnp.maximum(c0_ref[...] + c1_ref[...], 1.0)
    o_ref[...] = (q_ref[0] + q_ref[1]) / cnt[:, None] + z2_ref[...]


def _row_spec(nd=2):
    if nd == 1:
        return pl.BlockSpec((_BLK,), lambda i: (i,))
    return pl.BlockSpec((_BLK, _DH), lambda i: (i, 0))


def _full_spec(shape):
    nd = len(shape)
    return pl.BlockSpec(shape, lambda i: (0,) * nd)


def _par_spec():
    return pl.BlockSpec((_NC, _BLK, _DH), lambda i: (0, i, 0))


def _edge_pass_body(with_count, *refs):
    refs = list(refs)
    if with_count:
        (y_hbm, ei_hbm, out_hbm, cnt0_hbm, cnt1_hbm,
         ysh, acc, cntacc, sidx, didx) = refs[:10]
        bufs = refs[10:10 + 2 * _BK]
        (zbuf, zcnt, ones, psem, ssa, ssb) = refs[10 + 2 * _BK:16 + 2 * _BK]
        gsems = refs[16 + 2 * _BK:]
    else:
        (y_hbm, ei_hbm, out_hbm,
         ysh, acc, sidx, didx) = refs[:7]
        bufs = refs[7:7 + 2 * _BK]
        (zbuf, psem, ssa, ssb) = refs[7 + 2 * _BK:11 + 2 * _BK]
        gsems = refs[11 + 2 * _BK:]

    cid = lax.axis_index("c")
    tid = lax.axis_index("s")
    wid = tid * _NC + cid
    r0 = tid * _RPT
    e0 = wid * _CPW * _CH

    stg = pltpu.async_copy(y_hbm.at[pl.ds(r0, _RPT)],
                           ysh.at[pl.ds(r0, _RPT)], psem)
    sic = pltpu.async_copy(ei_hbm.at[0, pl.ds(e0, _CPW * _CH)],
                           sidx.at[pl.ds(0, _CPW * _CH)], psem)

    def _ldd(i, _):
        pltpu.async_copy(ei_hbm.at[1, pl.ds(e0 + i * _CH, _CH)],
                         didx.at[i], psem)
        return 0
    lax.fori_loop(0, _CPW, _ldd, 0)

    ex0 = (_CPW * _NW + wid) * _CH

    @pl.when(wid < _XTRA)
    def _():
        pltpu.async_copy(ei_hbm.at[0, pl.ds(ex0, _CH)],
                         sidx.at[pl.ds(_CPW * _CH, _CH)], psem)
        pltpu.async_copy(ei_hbm.at[1, pl.ds(ex0, _CH)], didx.at[_CPW], psem)

    zeros16 = jnp.zeros((16,), jnp.float32)

    def _zrow(i, _):
        zbuf[i, pl.ds(0, 16)] = zeros16
        zbuf[i, pl.ds(16, 16)] = zeros16
        return 0
    lax.fori_loop(0, _RPT, _zrow, 0)

    if with_count:
        def _zc(i, _):
            zcnt[pl.ds(i * 16, 16)] = zeros16
            return 0
        lax.fori_loop(0, _RPT // 16, _zc, 0)
        ones16 = jnp.ones((16,), jnp.float32)
        for i in range(_CH // 16):
            ones[pl.ds(i * 16, 16)] = ones16

    pltpu.sync_copy(zbuf, acc.at[pl.ds(r0, _RPT)])
    if with_count:
        pltpu.sync_copy(zcnt, cntacc.at[pl.ds(r0, _RPT)])

    stg.wait()
    sic.wait()

    def _ldw(i, _):
        pltpu.make_async_copy(ei_hbm.at[1, pl.ds(0, _CH)], didx.at[0],
                              psem).wait()
        return 0
    lax.fori_loop(0, _CPW, _ldw, 0)

    @pl.when(wid < _XTRA)
    def _():
        pltpu.make_async_copy(ei_hbm.at[0, pl.ds(0, _CH)],
                              sidx.at[pl.ds(_CPW * _CH, _CH)], psem).wait()
        pltpu.make_async_copy(ei_hbm.at[1, pl.ds(0, _CH)], didx.at[0],
                              psem).wait()

    plsc.subcore_barrier()

    def _gather(c, buf, sem):
        return pltpu.async_copy(ysh.at[sidx.at[pl.ds(c * _CH, _CH)]],
                                buf, sem)

    def _gwait(buf, sem):
        pltpu.make_async_copy(ysh.at[sidx.at[pl.ds(0, _CH)]], buf,
                              sem).wait()

    def _scat(c, buf, sem):
        pltpu.async_copy(buf, acc.at[didx.at[c]], sem, add=True)
        if with_count:
            pltpu.async_copy(ones, cntacc.at[didx.at[c]], sem, add=True)

    def _sdrain(buf, sem):
        pltpu.make_async_copy(buf, acc.at[didx.at[0]], sem).wait()
        if with_count:
            pltpu.make_async_copy(ones, cntacc.at[didx.at[0]], sem).wait()

    for b in range(_BK):
        _gather(b, bufs[b], gsems[b])
    for b in range(_BK):
        _gather(_BK + b, bufs[_BK + b], gsems[_BK + b])

    def _group(j, _):
        c = j * 2 * _BK
        for b in range(_BK):
            _gwait(bufs[b], gsems[b])
            _scat(c + b, bufs[b], ssa)
        for b in range(_BK):
            _sdrain(bufs[b], ssa)

        @pl.when(j < _NGRP - 1)
        def _():
            for b in range(_BK):
                _gather(c + 2 * _BK + b, bufs[b], gsems[b])

        for b in range(_BK):
            _gwait(bufs[_BK + b], gsems[_BK + b])
            _scat(c + _BK + b, bufs[_BK + b], ssb)
        for b in range(_BK):
            _sdrain(bufs[_BK + b], ssb)

        @pl.when(j < _NGRP - 1)
        def _():
            for b in range(_BK):
                _gather(c + 3 * _BK + b, bufs[_BK + b], gsems[_BK + b])
        return 0
    lax.fori_loop(0, _NGRP, _group, 0)

    @pl.when(wid < _XTRA)
    def _():
        _gather(_CPW, bufs[0], gsems[0])
        _gwait(bufs[0], gsems[0])
        _scat(_CPW, bufs[0], ssa)
        _sdrain(bufs[0], ssa)

    plsc.subcore_barrier()

    pltpu.sync_copy(acc.at[pl.ds(r0, _RPT)], out_hbm.at[cid, pl.ds(r0, _RPT)])
    if with_count:
        @pl.when(cid == 0)
        def _():
            pltpu.sync_copy(cntacc.at[pl.ds(r0, _RPT)],
                            cnt0_hbm.at[pl.ds(r0, _RPT)])

        @pl.when(cid == 1)
        def _():
            pltpu.sync_copy(cntacc.at[pl.ds(r0, _RPT)],
                            cnt1_hbm.at[pl.ds(r0, _RPT)])


def _make_edge_pass(with_count):
    out_type = [jax.ShapeDtypeStruct((_NC, _NP, _DH), jnp.float32)]
    if with_count:
        out_type.append(jax.ShapeDtypeStruct((_NP,), jnp.float32))
        out_type.append(jax.ShapeDtypeStruct((_NP,), jnp.float32))
    scratch = [
        pltpu.VMEM_SHARED((_NP, _DH), jnp.float32),
        pltpu.VMEM_SHARED((_NP, _DH), jnp.float32),
    ]
    if with_count:
        scratch.append(pltpu.VMEM_SHARED((_NP,), jnp.float32))
    scratch += [
        pltpu.VMEM(((_CPW + 1) * _CH,), jnp.int32),
        pltpu.VMEM((_CPW + 1, _CH), jnp.int32),
    ]
    scratch += [pltpu.VMEM((_CH, _DH), jnp.float32) for _ in range(2 * _BK)]
    scratch.append(pltpu.VMEM((_RPT, _DH), jnp.float32))
    if with_count:
        scratch.append(pltpu.VMEM((_RPT,), jnp.float32))
        scratch.append(pltpu.VMEM((_CH,), jnp.float32))
    scratch += [pltpu.SemaphoreType.DMA,
                pltpu.SemaphoreType.DMA,
                pltpu.SemaphoreType.DMA]
    scratch += [pltpu.SemaphoreType.DMA for _ in range(2 * _BK)]

    return pl.kernel(
        functools.partial(_edge_pass_body, with_count),
        out_type=out_type,
        mesh=plsc.VectorSubcoreMesh(core_axis_name="c", subcore_axis_name="s"),
        scratch_types=scratch,
        compiler_params=pltpu.CompilerParams(use_tc_tiling_on_sc=False),
    )


_edge_pass_l1 = _make_edge_pass(True)
_edge_pass_l2 = _make_edge_pass(False)


def _impl(x, edge_index, W1l, b1, W1r, W2l, b2, W2r):
    ei = edge_index.astype(jnp.int32)
    b1r = b1.reshape(1, _DH)
    b2r = b2.reshape(1, _DH)
    f32 = jnp.float32

    y1, z1 = pl.pallas_call(
        _lin_body,
        grid=(_GRID,),
        in_specs=[pl.BlockSpec((_BLK, _DIN), lambda i: (i, 0)),
                  _full_spec((_DIN, _DH)), _full_spec((_DIN, _DH)),
                  _full_spec((1, _DH))],
        out_specs=[_row_spec(), _row_spec()],
        out_shape=[jax.ShapeDtypeStruct((_NP, _DH), f32),
                   jax.ShapeDtypeStruct((_NP, _DH), f32)],
    )(x, W1l, W1r, b1r)

    p, cnt0, cnt1 = _edge_pass_l1(y1, ei)

    y2, z2 = pl.pallas_call(
        _mid_body,
        grid=(_GRID,),
        in_specs=[_par_spec(), _row_spec(1), _row_spec(1), _row_spec(),
                  _full_spec((_DH, _DH)), _full_spec((_DH, _DH)),
                  _full_spec((1, _DH))],
        out_specs=[_row_spec(), _row_spec()],
        out_shape=[jax.ShapeDtypeStruct((_NP, _DH), f32),
                   jax.ShapeDtypeStruct((_NP, _DH), f32)],
    )(p, cnt0, cnt1, z1, W2l, W2r, b2r)

    (q,) = _edge_pass_l2(y2, ei)

    out = pl.pallas_call(
        _fin_body,
        grid=(_GRID,),
        in_specs=[_par_spec(), _row_spec(1), _row_spec(1), _row_spec()],
        out_specs=_row_spec(),
        out_shape=jax.ShapeDtypeStruct((_N, _DH), f32),
    )(q, cnt0, cnt1, z2)
    return out


def kernel(x, edge_index, W1l, b1, W1r, W2l, b2, W2r):
    assert x.shape == (_N, _DIN) and edge_index.shape == (2, _E)
    return _impl(x, edge_index, W1l, b1, W1r, W2l, b2, W2r)


# speedup vs baseline: 24.6927x; 1.0728x over previous
import functools

import jax
import jax.numpy as jnp
from jax import lax
from jax.experimental import pallas as pl
from jax.experimental.pallas import tpu as pltpu
from jax.experimental.pallas import tpu_sc as plsc

_N = 10000
_E = 320000
_DIN = 128
_DH = 32
_NP = 10240

_NC = 2
_NS = 16
_NW = _NC * _NS
_CH = 128
_NCHUNK = _E // _CH
_CPW = _NCHUNK // _NW
_XTRA = _NCHUNK - _CPW * _NW
_BK = 3
_NGRP = _CPW // (2 * _BK)
_RPT = _NP // _NS

_BLK = 2048
_GRID = _NP // _BLK


def _lin_body(x_ref, wl_ref, wr_ref, b_ref, y_ref, z_ref):
    x = x_ref[...]
    y_ref[...] = jnp.dot(x, wl_ref[...], preferred_element_type=jnp.float32)
    z_ref[...] = (
        jnp.dot(x, wr_ref[...], preferred_element_type=jnp.float32) + b_ref[...]
    )


def _mid_body(p_ref, c0_ref, c1_ref, z1_ref, w2l_ref, w2r_ref, b2_ref,
              y2_ref, z2_ref):
    agg = p_ref[0] + p_ref[1]
    cnt = jnp.maximum(c0_ref[...] + c1_ref[...], 1.0)
    h = jnp.maximum(agg / cnt[:, None] + z1_ref[...], 0.0)
    y2_ref[...] = jnp.dot(h, w2l_ref[...], preferred_element_type=jnp.float32)
    z2_ref[...] = (
        jnp.dot(h, w2r_ref[...], preferred_element_type=jnp.float32) + b2_ref[...]
    )


def _fin_body(q_ref, c0_ref, c1_ref, z2_ref, o_ref):
    cnt = jnp.maximum(c0_ref[...] + c1_ref[...], 1.0)
    o_ref[...] = (q_ref[0] + q_ref[1]) / cnt[:, None] + z2_ref[...]


def _row_spec(nd=2):
    if nd == 1:
        return pl.BlockSpec((_BLK,), lambda i: (i,))
    return pl.BlockSpec((_BLK, _DH), lambda i: (i, 0))


def _full_spec(shape):
    nd = len(shape)
    return pl.BlockSpec(shape, lambda i: (0,) * nd)


def _par_spec():
    return pl.BlockSpec((_NC, _BLK, _DH), lambda i: (0, i, 0))


def _edge_pass_body(with_count, *refs):
    refs = list(refs)
    if with_count:
        (y_hbm, ei_hbm, out_hbm, cnt0_hbm, cnt1_hbm,
         acc, cntacc, sidx, didx) = refs[:9]
        bufs = refs[9:9 + 2 * _BK]
        (zbuf, zcnt, ones, psem, ssa, ssb) = refs[9 + 2 * _BK:15 + 2 * _BK]
        gsems = refs[15 + 2 * _BK:]
    else:
        (y_hbm, ei_hbm, out_hbm,
         acc, sidx, didx) = refs[:6]
        bufs = refs[6:6 + 2 * _BK]
        (zbuf, psem, ssa, ssb) = refs[6 + 2 * _BK:10 + 2 * _BK]
        gsems = refs[10 + 2 * _BK:]

    cid = lax.axis_index("c")
    tid = lax.axis_index("s")
    wid = tid * _NC + cid
    r0 = tid * _RPT
    e0 = wid * _CPW * _CH

    sic = pltpu.async_copy(ei_hbm.at[0, pl.ds(e0, _CPW * _CH)],
                           sidx.at[pl.ds(0, _CPW * _CH)], psem)

    def _ldd(i, _):
        pltpu.async_copy(ei_hbm.at[1, pl.ds(e0 + i * _CH, _CH)],
                         didx.at[i], psem)
        return 0
    lax.fori_loop(0, _CPW, _ldd, 0)

    ex0 = (_CPW * _NW + wid) * _CH

    @pl.when(wid < _XTRA)
    def _():
        pltpu.async_copy(ei_hbm.at[0, pl.ds(ex0, _CH)],
                         sidx.at[pl.ds(_CPW * _CH, _CH)], psem)
        pltpu.async_copy(ei_hbm.at[1, pl.ds(ex0, _CH)], didx.at[_CPW], psem)

    zeros16 = jnp.zeros((16,), jnp.float32)

    def _zrow(i, _):
        zbuf[i, pl.ds(0, 16)] = zeros16
        zbuf[i, pl.ds(16, 16)] = zeros16
        return 0
    lax.fori_loop(0, _RPT, _zrow, 0)

    if with_count:
        def _zc(i, _):
            zcnt[pl.ds(i * 16, 16)] = zeros16
            return 0
        lax.fori_loop(0, _RPT // 16, _zc, 0)
        ones16 = jnp.ones((16,), jnp.float32)
        for i in range(_CH // 16):
            ones[pl.ds(i * 16, 16)] = ones16

    pltpu.sync_copy(zbuf, acc.at[pl.ds(r0, _RPT)])
    if with_count:
        pltpu.sync_copy(zcnt, cntacc.at[pl.ds(r0, _RPT)])

    sic.wait()

    def _ldw(i, _):
        pltpu.make_async_copy(ei_hbm.at[1, pl.ds(0, _CH)], didx.at[0],
                              psem).wait()
        return 0
    lax.fori_loop(0, _CPW, _ldw, 0)

    @pl.when(wid < _XTRA)
    def _():
        pltpu.make_async_copy(ei_hbm.at[0, pl.ds(0, _CH)],
                              sidx.at[pl.ds(_CPW * _CH, _CH)], psem).wait()
        pltpu.make_async_copy(ei_hbm.at[1, pl.ds(0, _CH)], didx.at[0],
                              psem).wait()

    plsc.subcore_barrier()

    def _gather(c, buf, sem):
        return pltpu.async_copy(y_hbm.at[sidx.at[pl.ds(c * _CH, _CH)]],
                                buf, sem)

    def _gwait(buf, sem):
        pltpu.make_async_copy(y_hbm.at[sidx.at[pl.ds(0, _CH)]], buf,
                              sem).wait()

    def _scat(c, buf, sem):
        pltpu.async_copy(buf, acc.at[didx.at[c]], sem, add=True)
        if with_count:
            pltpu.async_copy(ones, cntacc.at[didx.at[c]], sem, add=True)

    def _sdrain(buf, sem):
        pltpu.make_async_copy(buf, acc.at[didx.at[0]], sem).wait()
        if with_count:
            pltpu.make_async_copy(ones, cntacc.at[didx.at[0]], sem).wait()

    for b in range(_BK):
        _gather(b, bufs[b], gsems[b])
    for b in range(_BK):
        _gather(_BK + b, bufs[_BK + b], gsems[_BK + b])

    def _group(j, _):
        c = j * 2 * _BK
        for b in range(_BK):
            _gwait(bufs[b], gsems[b])
            _scat(c + b, bufs[b], ssa)
        for b in range(_BK):
            _sdrain(bufs[b], ssa)

        @pl.when(j < _NGRP - 1)
        def _():
            for b in range(_BK):
                _gather(c + 2 * _BK + b, bufs[b], gsems[b])

        for b in range(_BK):
            _gwait(bufs[_BK + b], gsems[_BK + b])
            _scat(c + _BK + b, bufs[_BK + b], ssb)
        for b in range(_BK):
            _sdrain(bufs[_BK + b], ssb)

        @pl.when(j < _NGRP - 1)
        def _():
            for b in range(_BK):
                _gather(c + 3 * _BK + b, bufs[_BK + b], gsems[_BK + b])
        return 0
    lax.fori_loop(0, _NGRP, _group, 0)

    @pl.when(wid < _XTRA)
    def _():
        _gather(_CPW, bufs[0], gsems[0])
        _gwait(bufs[0], gsems[0])
        _scat(_CPW, bufs[0], ssa)
        _sdrain(bufs[0], ssa)

    plsc.subcore_barrier()

    pltpu.sync_copy(acc.at[pl.ds(r0, _RPT)], out_hbm.at[cid, pl.ds(r0, _RPT)])
    if with_count:
        @pl.when(cid == 0)
        def _():
            pltpu.sync_copy(cntacc.at[pl.ds(r0, _RPT)],
                            cnt0_hbm.at[pl.ds(r0, _RPT)])

        @pl.when(cid == 1)
        def _():
            pltpu.sync_copy(cntacc.at[pl.ds(r0, _RPT)],
                            cnt1_hbm.at[pl.ds(r0, _RPT)])


def _make_edge_pass(with_count):
    out_type = [jax.ShapeDtypeStruct((_NC, _NP, _DH), jnp.float32)]
    if with_count:
        out_type.append(jax.ShapeDtypeStruct((_NP,), jnp.float32))
        out_type.append(jax.ShapeDtypeStruct((_NP,), jnp.float32))
    scratch = [
        pltpu.VMEM_SHARED((_NP, _DH), jnp.float32),
    ]
    if with_count:
        scratch.append(pltpu.VMEM_SHARED((_NP,), jnp.float32))
    scratch += [
        pltpu.VMEM(((_CPW + 1) * _CH,), jnp.int32),
        pltpu.VMEM((_CPW + 1, _CH), jnp.int32),
    ]
    scratch += [pltpu.VMEM((_CH, _DH), jnp.float32) for _ in range(2 * _BK)]
    scratch.append(pltpu.VMEM((_RPT, _DH), jnp.float32))
    if with_count:
        scratch.append(pltpu.VMEM((_RPT,), jnp.float32))
        scratch.append(pltpu.VMEM((_CH,), jnp.float32))
    scratch += [pltpu.SemaphoreType.DMA,
                pltpu.SemaphoreType.DMA,
                pltpu.SemaphoreType.DMA]
    scratch += [pltpu.SemaphoreType.DMA for _ in range(2 * _BK)]

    return pl.kernel(
        functools.partial(_edge_pass_body, with_count),
        out_type=out_type,
        mesh=plsc.VectorSubcoreMesh(core_axis_name="c", subcore_axis_name="s"),
        scratch_types=scratch,
        compiler_params=pltpu.CompilerParams(use_tc_tiling_on_sc=False),
    )


_edge_pass_l1 = _make_edge_pass(True)
_edge_pass_l2 = _make_edge_pass(False)


def _impl(x, edge_index, W1l, b1, W1r, W2l, b2, W2r):
    ei = edge_index.astype(jnp.int32)
    b1r = b1.reshape(1, _DH)
    b2r = b2.reshape(1, _DH)
    f32 = jnp.float32

    y1, z1 = pl.pallas_call(
        _lin_body,
        grid=(_GRID,),
        in_specs=[pl.BlockSpec((_BLK, _DIN), lambda i: (i, 0)),
                  _full_spec((_DIN, _DH)), _full_spec((_DIN, _DH)),
                  _full_spec((1, _DH))],
        out_specs=[_row_spec(), _row_spec()],
        out_shape=[jax.ShapeDtypeStruct((_NP, _DH), f32),
                   jax.ShapeDtypeStruct((_NP, _DH), f32)],
    )(x, W1l, W1r, b1r)

    p, cnt0, cnt1 = _edge_pass_l1(y1, ei)

    y2, z2 = pl.pallas_call(
        _mid_body,
        grid=(_GRID,),
        in_specs=[_par_spec(), _row_spec(1), _row_spec(1), _row_spec(),
                  _full_spec((_DH, _DH)), _full_spec((_DH, _DH)),
                  _full_spec((1, _DH))],
        out_specs=[_row_spec(), _row_spec()],
        out_shape=[jax.ShapeDtypeStruct((_NP, _DH), f32),
                   jax.ShapeDtypeStruct((_NP, _DH), f32)],
    )(p, cnt0, cnt1, z1, W2l, W2r, b2r)

    (q,) = _edge_pass_l2(y2, ei)

    out = pl.pallas_call(
        _fin_body,
        grid=(_GRID,),
        in_specs=[_par_spec(), _row_spec(1), _row_spec(1), _row_spec()],
        out_specs=_row_spec(),
        out_shape=jax.ShapeDtypeStruct((_N, _DH), f32),
    )(q, cnt0, cnt1, z2)
    return out


def kernel(x, edge_index, W1l, b1, W1r, W2l, b2, W2r):
    assert x.shape == (_N, _DIN) and edge_index.shape == (2, _E)
    return _impl(x, edge_index, W1l, b1, W1r, W2l, b2, W2r)


# speedup vs baseline: 29.4977x; 1.1946x over previous
import functools

import jax
import jax.numpy as jnp
from jax import lax
from jax.experimental import pallas as pl
from jax.experimental.pallas import tpu as pltpu
from jax.experimental.pallas import tpu_sc as plsc

_N = 10000
_E = 320000
_DIN = 128
_DH = 32
_NP = 10240

_NC = 2
_NS = 16
_NW = _NC * _NS
_CH = 128
_NCHUNK = _E // _CH
_CPW = _NCHUNK // _NW
_XTRA = _NCHUNK - _CPW * _NW
_BK = 3
_NGRP = _CPW // (2 * _BK)
_RPT = _NP // _NS

_BLK = 2048
_GRID = _NP // _BLK


_PB = _BLK // 4
_FB = _BLK * _DH


def _lin_body(x_ref, wl_ref, wr_ref, b_ref, y_ref, z_ref):
    xr = x_ref[...].reshape(_PB, 4, _DIN)
    ys, zs = [], []
    for m in range(4):
        xm = xr[:, m, :]
        ys.append(jnp.dot(xm, wl_ref[...], preferred_element_type=jnp.float32))
        zs.append(jnp.dot(xm, wr_ref[...], preferred_element_type=jnp.float32))
    yp = jnp.concatenate(ys, axis=1)
    zp = jnp.concatenate(zs, axis=1) + b_ref[...]
    y_ref[...] = yp.reshape(_FB)
    z_ref[...] = zp


def _recip_packed(c0, c1, r4):
    cnt = jnp.maximum(c0 + c1, 1.0)
    return jnp.dot(1.0 / cnt, r4, preferred_element_type=jnp.float32)


def _mid_body(p0_ref, p1_ref, c0_ref, c1_ref, z1_ref, w2l_ref, w2r_ref,
              b2_ref, r4_ref, y2_ref, z2_ref):
    aggp = (p0_ref[...] + p1_ref[...]).reshape(_PB, 128)
    rp = _recip_packed(c0_ref[...], c1_ref[...], r4_ref[...])
    hp = jnp.maximum(aggp * rp + z1_ref[...], 0.0)
    y2_ref[...] = jnp.dot(
        hp, w2l_ref[...], preferred_element_type=jnp.float32).reshape(_FB)
    z2_ref[...] = (
        jnp.dot(hp, w2r_ref[...], preferred_element_type=jnp.float32)
        + b2_ref[...])


def _fin_body(q0_ref, q1_ref, c0_ref, c1_ref, z2_ref, r4_ref, o_ref):
    qp = (q0_ref[...] + q1_ref[...]).reshape(_PB, 128)
    rp = _recip_packed(c0_ref[...], c1_ref[...], r4_ref[...])
    o_ref[...] = (qp * rp + z2_ref[...]).reshape(_FB)


def _flat_spec(off=0):
    if off:
        return pl.BlockSpec((_FB,), lambda i: (i + off,))
    return pl.BlockSpec((_FB,), lambda i: (i,))


def _pk_spec():
    return pl.BlockSpec((_PB, 128), lambda i: (i, 0))


def _cnt_spec():
    return pl.BlockSpec((_PB, 4), lambda i: (i, 0))


def _full_spec(shape):
    nd = len(shape)
    return pl.BlockSpec(shape, lambda i: (0,) * nd)


def _edge_pass_body(with_count, *refs):
    refs = list(refs)
    if with_count:
        (y_hbm, ei_hbm, out_hbm, cnt0_hbm, cnt1_hbm,
         acc, cntacc, sidx, didx) = refs[:9]
        bufs = refs[9:9 + 2 * _BK]
        (zbuf, zcnt, ones, psem, ssa, ssb) = refs[9 + 2 * _BK:15 + 2 * _BK]
        gsems = refs[15 + 2 * _BK:]
    else:
        (y_hbm, ei_hbm, out_hbm,
         acc, sidx, didx) = refs[:6]
        bufs = refs[6:6 + 2 * _BK]
        (zbuf, psem, ssa, ssb) = refs[6 + 2 * _BK:10 + 2 * _BK]
        gsems = refs[10 + 2 * _BK:]

    cid = lax.axis_index("c")
    tid = lax.axis_index("s")
    wid = tid * _NC + cid
    r0 = tid * _RPT
    e0 = wid * _CPW * _CH

    sic = pltpu.async_copy(ei_hbm.at[0, pl.ds(e0, _CPW * _CH)],
                           sidx.at[pl.ds(0, _CPW * _CH)], psem)

    def _ldd(i, _):
        pltpu.async_copy(ei_hbm.at[1, pl.ds(e0 + i * _CH, _CH)],
                         didx.at[i], psem)
        return 0
    lax.fori_loop(0, _CPW, _ldd, 0)

    ex0 = (_CPW * _NW + wid) * _CH

    @pl.when(wid < _XTRA)
    def _():
        pltpu.async_copy(ei_hbm.at[0, pl.ds(ex0, _CH)],
                         sidx.at[pl.ds(_CPW * _CH, _CH)], psem)
        pltpu.async_copy(ei_hbm.at[1, pl.ds(ex0, _CH)], didx.at[_CPW], psem)

    zeros16 = jnp.zeros((16,), jnp.float32)

    def _zrow(i, _):
        zbuf[i, pl.ds(0, 16)] = zeros16
        zbuf[i, pl.ds(16, 16)] = zeros16
        return 0
    lax.fori_loop(0, _RPT, _zrow, 0)

    if with_count:
        def _zc(i, _):
            zcnt[pl.ds(i * 16, 16)] = zeros16
            return 0
        lax.fori_loop(0, _RPT // 16, _zc, 0)
        ones16 = jnp.ones((16,), jnp.float32)
        for i in range(_CH // 16):
            ones[pl.ds(i * 16, 16)] = ones16

    pltpu.sync_copy(zbuf, acc.at[pl.ds(r0, _RPT)])
    if with_count:
        pltpu.sync_copy(zcnt, cntacc.at[pl.ds(r0, _RPT)])

    sic.wait()

    def _ldw(i, _):
        pltpu.make_async_copy(ei_hbm.at[1, pl.ds(0, _CH)], didx.at[0],
                              psem).wait()
        return 0
    lax.fori_loop(0, _CPW, _ldw, 0)

    @pl.when(wid < _XTRA)
    def _():
        pltpu.make_async_copy(ei_hbm.at[0, pl.ds(0, _CH)],
                              sidx.at[pl.ds(_CPW * _CH, _CH)], psem).wait()
        pltpu.make_async_copy(ei_hbm.at[1, pl.ds(0, _CH)], didx.at[0],
                              psem).wait()

    plsc.subcore_barrier()

    def _gather(c, buf, sem):
        return pltpu.async_copy(y_hbm.at[sidx.at[pl.ds(c * _CH, _CH)]],
                                buf, sem)

    def _gwait(buf, sem):
        pltpu.make_async_copy(y_hbm.at[sidx.at[pl.ds(0, _CH)]], buf,
                              sem).wait()

    def _scat(c, buf, sem):
        pltpu.async_copy(buf, acc.at[didx.at[c]], sem, add=True)
        if with_count:
            pltpu.async_copy(ones, cntacc.at[didx.at[c]], sem, add=True)

    def _sdrain(buf, sem):
        pltpu.make_async_copy(buf, acc.at[didx.at[0]], sem).wait()
        if with_count:
            pltpu.make_async_copy(ones, cntacc.at[didx.at[0]], sem).wait()

    for b in range(_BK):
        _gather(b, bufs[b], gsems[b])
    for b in range(_BK):
        _gather(_BK + b, bufs[_BK + b], gsems[_BK + b])

    def _group(j, _):
        c = j * 2 * _BK
        for b in range(_BK):
            _gwait(bufs[b], gsems[b])
            _scat(c + b, bufs[b], ssa)
        for b in range(_BK):
            _sdrain(bufs[b], ssa)

        @pl.when(j < _NGRP - 1)
        def _():
            for b in range(_BK):
                _gather(c + 2 * _BK + b, bufs[b], gsems[b])

        for b in range(_BK):
            _gwait(bufs[_BK + b], gsems[_BK + b])
            _scat(c + _BK + b, bufs[_BK + b], ssb)
        for b in range(_BK):
            _sdrain(bufs[_BK + b], ssb)

        @pl.when(j < _NGRP - 1)
        def _():
            for b in range(_BK):
                _gather(c + 3 * _BK + b, bufs[_BK + b], gsems[_BK + b])
        return 0
    lax.fori_loop(0, _NGRP, _group, 0)

    @pl.when(wid < _XTRA)
    def _():
        _gather(_CPW, bufs[0], gsems[0])
        _gwait(bufs[0], gsems[0])
        _scat(_CPW, bufs[0], ssa)
        _sdrain(bufs[0], ssa)

    plsc.subcore_barrier()

    pltpu.sync_copy(acc.at[pl.ds(r0, _RPT)], out_hbm.at[cid, pl.ds(r0, _RPT)])
    if with_count:
        @pl.when(cid == 0)
        def _():
            pltpu.sync_copy(cntacc.at[pl.ds(r0, _RPT)],
                            cnt0_hbm.at[pl.ds(r0, _RPT)])

        @pl.when(cid == 1)
        def _():
            pltpu.sync_copy(cntacc.at[pl.ds(r0, _RPT)],
                            cnt1_hbm.at[pl.ds(r0, _RPT)])


def _make_edge_pass(with_count):
    out_type = [jax.ShapeDtypeStruct((_NC, _NP, _DH), jnp.float32)]
    if with_count:
        out_type.append(jax.ShapeDtypeStruct((_NP,), jnp.float32))
        out_type.append(jax.ShapeDtypeStruct((_NP,), jnp.float32))
    scratch = [
        pltpu.VMEM_SHARED((_NP, _DH), jnp.float32),
    ]
    if with_count:
        scratch.append(pltpu.VMEM_SHARED((_NP,), jnp.float32))
    scratch += [
        pltpu.VMEM(((_CPW + 1) * _CH,), jnp.int32),
        pltpu.VMEM((_CPW + 1, _CH), jnp.int32),
    ]
    scratch += [pltpu.VMEM((_CH, _DH), jnp.float32) for _ in range(2 * _BK)]
    scratch.append(pltpu.VMEM((_RPT, _DH), jnp.float32))
    if with_count:
        scratch.append(pltpu.VMEM((_RPT,), jnp.float32))
        scratch.append(pltpu.VMEM((_CH,), jnp.float32))
    scratch += [pltpu.SemaphoreType.DMA,
                pltpu.SemaphoreType.DMA,
                pltpu.SemaphoreType.DMA]
    scratch += [pltpu.SemaphoreType.DMA for _ in range(2 * _BK)]

    return pl.kernel(
        functools.partial(_edge_pass_body, with_count),
        out_type=out_type,
        mesh=plsc.VectorSubcoreMesh(core_axis_name="c", subcore_axis_name="s"),
        scratch_types=scratch,
        compiler_params=pltpu.CompilerParams(use_tc_tiling_on_sc=False),
    )


_edge_pass_l1 = _make_edge_pass(True)
_edge_pass_l2 = _make_edge_pass(False)


def _impl(x, edge_index, W1l, b1, W1r, W2l, b2, W2r):
    f32 = jnp.float32
    ei = edge_index.astype(jnp.int32)
    eye4 = jnp.eye(4, dtype=f32)
    b1p = jnp.tile(b1, 4).reshape(1, 128)
    b2p = jnp.tile(b2, 4).reshape(1, 128)
    W2lB = jnp.kron(eye4, W2l)
    W2rB = jnp.kron(eye4, W2r)
    R4 = jnp.kron(eye4, jnp.ones((1, _DH), f32))

    y1f, z1 = pl.pallas_call(
        _lin_body,
        grid=(_GRID,),
        in_specs=[pl.BlockSpec((_BLK, _DIN), lambda i: (i, 0)),
                  _full_spec((_DIN, _DH)), _full_spec((_DIN, _DH)),
                  _full_spec((1, 128))],
        out_specs=[_flat_spec(), _pk_spec()],
        out_shape=[jax.ShapeDtypeStruct((_NP * _DH,), f32),
                   jax.ShapeDtypeStruct((_NP // 4, 128), f32)],
    )(x, W1l, W1r, b1p)

    p, cnt0, cnt1 = _edge_pass_l1(y1f.reshape(_NP, _DH), ei)
    pf = p.reshape(_NC * _NP * _DH)
    c02, c12 = cnt0.reshape(_NP // 4, 4), cnt1.reshape(_NP // 4, 4)

    y2f, z2 = pl.pallas_call(
        _mid_body,
        grid=(_GRID,),
        in_specs=[_flat_spec(), _flat_spec(_GRID), _cnt_spec(), _cnt_spec(),
                  _pk_spec(), _full_spec((128, 128)), _full_spec((128, 128)),
                  _full_spec((1, 128)), _full_spec((4, 128))],
        out_specs=[_flat_spec(), _pk_spec()],
        out_shape=[jax.ShapeDtypeStruct((_NP * _DH,), f32),
                   jax.ShapeDtypeStruct((_NP // 4, 128), f32)],
    )(pf, pf, c02, c12, z1, W2lB, W2rB, b2p, R4)

    (q,) = _edge_pass_l2(y2f.reshape(_NP, _DH), ei)
    qf = q.reshape(_NC * _NP * _DH)

    outf = pl.pallas_call(
        _fin_body,
        grid=(_GRID,),
        in_specs=[_flat_spec(), _flat_spec(_GRID), _cnt_spec(), _cnt_spec(),
                  _pk_spec(), _full_spec((4, 128))],
        out_specs=_flat_spec(),
        out_shape=jax.ShapeDtypeStruct((_NP * _DH,), f32),
    )(qf, qf, c02, c12, z2, R4)
    return outf[:_N * _DH].reshape(_N, _DH)


def kernel(x, edge_index, W1l, b1, W1r, W2l, b2, W2r):
    assert x.shape == (_N, _DIN) and edge_index.shape == (2, _E)
    return _impl(x, edge_index, W1l, b1, W1r, W2l, b2, W2r)


# speedup vs baseline: 29.5468x; 1.0017x over previous
import functools

import jax
import jax.numpy as jnp
from jax import lax
from jax.experimental import pallas as pl
from jax.experimental.pallas import tpu as pltpu
from jax.experimental.pallas import tpu_sc as plsc

_N = 10000
_E = 320000
_DIN = 128
_DH = 32
_NP = 10240

_NC = 2
_NS = 16
_NW = _NC * _NS
_CH = 128
_NCHUNK = _E // _CH
_CPW = _NCHUNK // _NW
_XTRA = _NCHUNK - _CPW * _NW
_BK = 3
_NGRP = _CPW // (2 * _BK)
_RPT = _NP // _NS

_BLK = 2048
_GRID = _NP // _BLK


_PB = _BLK // 4
_FB = _BLK * _DH


def _lin_body(x_ref, wl_ref, wr_ref, b_ref, y_ref, z_ref):
    xr = x_ref[...].reshape(_PB, 4, _DIN)
    ys, zs = [], []
    for m in range(4):
        xm = xr[:, m, :]
        ys.append(jnp.dot(xm, wl_ref[...], preferred_element_type=jnp.float32))
        zs.append(jnp.dot(xm, wr_ref[...], preferred_element_type=jnp.float32))
    yp = jnp.concatenate(ys, axis=1)
    zp = jnp.concatenate(zs, axis=1) + b_ref[...]
    y_ref[...] = yp.reshape(_FB)
    z_ref[...] = zp


def _recip_packed(c0, c1, r4):
    cnt = jnp.maximum(c0 + c1, 1.0)
    return jnp.dot(1.0 / cnt, r4, preferred_element_type=jnp.float32)


def _mid_body(p0_ref, p1_ref, c0_ref, c1_ref, z1_ref, w2l_ref, w2r_ref,
              b2_ref, r4_ref, y2_ref, z2_ref):
    aggp = (p0_ref[...] + p1_ref[...]).reshape(_PB, 128)
    rp = _recip_packed(c0_ref[...], c1_ref[...], r4_ref[...])
    hp = jnp.maximum(aggp * rp + z1_ref[...], 0.0)
    y2_ref[...] = jnp.dot(
        hp, w2l_ref[...], preferred_element_type=jnp.float32).reshape(_FB)
    z2_ref[...] = (
        jnp.dot(hp, w2r_ref[...], preferred_element_type=jnp.float32)
        + b2_ref[...])


def _fin_body(q0_ref, q1_ref, c0_ref, c1_ref, z2_ref, r4_ref, o_ref):
    qp = (q0_ref[...] + q1_ref[...]).reshape(_PB, 128)
    rp = _recip_packed(c0_ref[...], c1_ref[...], r4_ref[...])
    o_ref[...] = (qp * rp + z2_ref[...]).reshape(_FB)


def _flat_spec(off=0):
    if off:
        return pl.BlockSpec((_FB,), lambda i: (i + off,))
    return pl.BlockSpec((_FB,), lambda i: (i,))


def _pk_spec():
    return pl.BlockSpec((_PB, 128), lambda i: (i, 0))


def _cnt_spec():
    return pl.BlockSpec((_PB, 4), lambda i: (i, 0))


def _full_spec(shape):
    nd = len(shape)
    return pl.BlockSpec(shape, lambda i: (0,) * nd)


def _edge_pass_body(with_count, *refs):
    refs = list(refs)
    if with_count:
        (y_hbm, ei_hbm, out_hbm, cnt0_hbm, cnt1_hbm,
         acc, cntacc, sidx, didx) = refs[:9]
        bufs = refs[9:9 + 2 * _BK]
        (zbuf, zcnt, ones, psem, ssa, ssb) = refs[9 + 2 * _BK:15 + 2 * _BK]
        gsems = refs[15 + 2 * _BK:]
    else:
        (y_hbm, ei_hbm, out_hbm,
         acc, sidx, didx) = refs[:6]
        bufs = refs[6:6 + 2 * _BK]
        (zbuf, psem, ssa, ssb) = refs[6 + 2 * _BK:10 + 2 * _BK]
        gsems = refs[10 + 2 * _BK:]

    cid = lax.axis_index("c")
    tid = lax.axis_index("s")
    wid = tid * _NC + cid
    r0 = tid * _RPT
    e0 = wid * _CPW * _CH

    sic = pltpu.async_copy(ei_hbm.at[0, pl.ds(e0, _CPW * _CH)],
                           sidx.at[pl.ds(0, _CPW * _CH)], psem)

    def _ldd(i, _):
        pltpu.async_copy(ei_hbm.at[1, pl.ds(e0 + i * _CH, _CH)],
                         didx.at[i], psem)
        return 0
    lax.fori_loop(0, _CPW, _ldd, 0)

    ex0 = (_CPW * _NW + wid) * _CH

    @pl.when(wid < _XTRA)
    def _():
        pltpu.async_copy(ei_hbm.at[0, pl.ds(ex0, _CH)],
                         sidx.at[pl.ds(_CPW * _CH, _CH)], psem)
        pltpu.async_copy(ei_hbm.at[1, pl.ds(ex0, _CH)], didx.at[_CPW], psem)

    zeros16 = jnp.zeros((16,), jnp.float32)

    def _zrow(i, _):
        zbuf[i, pl.ds(0, 16)] = zeros16
        zbuf[i, pl.ds(16, 16)] = zeros16
        return 0
    lax.fori_loop(0, _RPT, _zrow, 0)

    if with_count:
        def _zc(i, _):
            zcnt[pl.ds(i * 16, 16)] = zeros16
            return 0
        lax.fori_loop(0, _RPT // 16, _zc, 0)
        ones16 = jnp.ones((16,), jnp.float32)
        for i in range(_CH // 16):
            ones[pl.ds(i * 16, 16)] = ones16

    pltpu.sync_copy(zbuf, acc.at[pl.ds(r0, _RPT)])
    if with_count:
        pltpu.sync_copy(zcnt, cntacc.at[pl.ds(r0, _RPT)])

    sic.wait()

    def _ldw(i, _):
        pltpu.make_async_copy(ei_hbm.at[1, pl.ds(0, _CH)], didx.at[0],
                              psem).wait()
        return 0
    lax.fori_loop(0, _CPW, _ldw, 0)

    @pl.when(wid < _XTRA)
    def _():
        pltpu.make_async_copy(ei_hbm.at[0, pl.ds(0, _CH)],
                              sidx.at[pl.ds(_CPW * _CH, _CH)], psem).wait()
        pltpu.make_async_copy(ei_hbm.at[1, pl.ds(0, _CH)], didx.at[0],
                              psem).wait()

    plsc.subcore_barrier()

    def _gather(c, buf, sem):
        return pltpu.async_copy(y_hbm.at[sidx.at[pl.ds(c * _CH, _CH)]],
                                buf, sem)

    def _gwait(buf, sem):
        pltpu.make_async_copy(y_hbm.at[sidx.at[pl.ds(0, _CH)]], buf,
                              sem).wait()

    def _scat(c, buf, sem):
        pltpu.async_copy(buf, acc.at[didx.at[c]], sem, add=True)
        if with_count:
            pltpu.async_copy(ones, cntacc.at[didx.at[c]], sem, add=True)

    def _sdrain(buf, sem):
        pltpu.make_async_copy(buf, acc.at[didx.at[0]], sem).wait()
        if with_count:
            pltpu.make_async_copy(ones, cntacc.at[didx.at[0]], sem).wait()

    for b in range(_BK):
        _gather(b, bufs[b], gsems[b])
    for b in range(_BK):
        _gather(_BK + b, bufs[_BK + b], gsems[_BK + b])

    def _group(j, _):
        c = j * 2 * _BK
        for b in range(_BK):
            _gwait(bufs[b], gsems[b])
            _scat(c + b, bufs[b], ssa)
        for b in range(_BK):
            _sdrain(bufs[b], ssa)

        @pl.when(j < _NGRP - 1)
        def _():
            for b in range(_BK):
                _gather(c + 2 * _BK + b, bufs[b], gsems[b])

        for b in range(_BK):
            _gwait(bufs[_BK + b], gsems[_BK + b])
            _scat(c + _BK + b, bufs[_BK + b], ssb)
        for b in range(_BK):
            _sdrain(bufs[_BK + b], ssb)

        @pl.when(j < _NGRP - 1)
        def _():
            for b in range(_BK):
                _gather(c + 3 * _BK + b, bufs[_BK + b], gsems[_BK + b])
        return 0
    lax.fori_loop(0, _NGRP, _group, 0)

    @pl.when(wid < _XTRA)
    def _():
        _gather(_CPW, bufs[0], gsems[0])
        _gwait(bufs[0], gsems[0])
        _scat(_CPW, bufs[0], ssa)
        _sdrain(bufs[0], ssa)

    plsc.subcore_barrier()

    pltpu.sync_copy(acc.at[pl.ds(r0, _RPT)], out_hbm.at[cid, pl.ds(r0, _RPT)])
    if with_count:
        @pl.when(cid == 0)
        def _():
            pltpu.sync_copy(cntacc.at[pl.ds(r0, _RPT)],
                            cnt0_hbm.at[pl.ds(r0, _RPT)])

        @pl.when(cid == 1)
        def _():
            pltpu.sync_copy(cntacc.at[pl.ds(r0, _RPT)],
                            cnt1_hbm.at[pl.ds(r0, _RPT)])


def _make_edge_pass(with_count):
    out_type = [jax.ShapeDtypeStruct((_NC, _NP, _DH), jnp.float32)]
    if with_count:
        out_type.append(jax.ShapeDtypeStruct((_NP,), jnp.float32))
        out_type.append(jax.ShapeDtypeStruct((_NP,), jnp.float32))
    scratch = [
        pltpu.VMEM_SHARED((_NP, _DH), jnp.float32),
    ]
    if with_count:
        scratch.append(pltpu.VMEM_SHARED((_NP,), jnp.float32))
    scratch += [
        pltpu.VMEM(((_CPW + 1) * _CH,), jnp.int32),
        pltpu.VMEM((_CPW + 1, _CH), jnp.int32),
    ]
    scratch += [pltpu.VMEM((_CH, _DH), jnp.float32) for _ in range(2 * _BK)]
    scratch.append(pltpu.VMEM((_RPT, _DH), jnp.float32))
    if with_count:
        scratch.append(pltpu.VMEM((_RPT,), jnp.float32))
        scratch.append(pltpu.VMEM((_CH,), jnp.float32))
    scratch += [pltpu.SemaphoreType.DMA,
                pltpu.SemaphoreType.DMA,
                pltpu.SemaphoreType.DMA]
    scratch += [pltpu.SemaphoreType.DMA for _ in range(2 * _BK)]

    return pl.kernel(
        functools.partial(_edge_pass_body, with_count),
        out_type=out_type,
        mesh=plsc.VectorSubcoreMesh(core_axis_name="c", subcore_axis_name="s"),
        scratch_types=scratch,
        compiler_params=pltpu.CompilerParams(use_tc_tiling_on_sc=False),
    )


_edge_pass_l1 = _make_edge_pass(True)
_edge_pass_l2 = _make_edge_pass(False)


def _impl(x, edge_index, W1l, b1, W1r, W2l, b2, W2r):
    f32 = jnp.float32
    ei = edge_index.astype(jnp.int32)
    eye4 = jnp.eye(4, dtype=f32)
    b1p = jnp.tile(b1, 4).reshape(1, 128)
    b2p = jnp.tile(b2, 4).reshape(1, 128)
    W2lB = jnp.kron(eye4, W2l)
    W2rB = jnp.kron(eye4, W2r)
    R4 = jnp.kron(eye4, jnp.ones((1, _DH), f32))

    y1f, z1 = pl.pallas_call(
        _lin_body,
        grid=(_GRID,),
        in_specs=[pl.BlockSpec((_BLK, _DIN), lambda i: (i, 0)),
                  _full_spec((_DIN, _DH)), _full_spec((_DIN, _DH)),
                  _full_spec((1, 128))],
        out_specs=[_flat_spec(), _pk_spec()],
        out_shape=[jax.ShapeDtypeStruct((_NP * _DH,), f32),
                   jax.ShapeDtypeStruct((_NP // 4, 128), f32)],
    )(x, W1l, W1r, b1p)

    p, cnt0, cnt1 = _edge_pass_l1(y1f.reshape(_NP, _DH), ei)
    pf = p.reshape(_NC * _NP * _DH)
    c02, c12 = cnt0.reshape(_NP // 4, 4), cnt1.reshape(_NP // 4, 4)

    y2f, z2 = pl.pallas_call(
        _mid_body,
        grid=(_GRID,),
        in_specs=[_flat_spec(), _flat_spec(_GRID), _cnt_spec(), _cnt_spec(),
                  _pk_spec(), _full_spec((128, 128)), _full_spec((128, 128)),
                  _full_spec((1, 128)), _full_spec((4, 128))],
        out_specs=[_flat_spec(), _pk_spec()],
        out_shape=[jax.ShapeDtypeStruct((_NP * _DH,), f32),
                   jax.ShapeDtypeStruct((_NP // 4, 128), f32)],
    )(pf, pf, c02, c12, z1, W2lB, W2rB, b2p, R4)

    (q,) = _edge_pass_l2(y2f.reshape(_NP, _DH), ei)
    qf = q.reshape(_NC * _NP * _DH)

    outf = pl.pallas_call(
        _fin_body,
        grid=(_GRID,),
        in_specs=[_flat_spec(), _flat_spec(_GRID), _cnt_spec(), _cnt_spec(),
                  _pk_spec(), _full_spec((4, 128))],
        out_specs=_flat_spec(),
        out_shape=jax.ShapeDtypeStruct((_NP * _DH,), f32),
    )(qf, qf, c02, c12, z2, R4)
    return outf.reshape(_NP, _DH)[:_N]


def kernel(x, edge_index, W1l, b1, W1r, W2l, b2, W2r):
    assert x.shape == (_N, _DIN) and edge_index.shape == (2, _E)
    return _impl(x, edge_index, W1l, b1, W1r, W2l, b2, W2r)
